# Initial kernel scaffold; baseline (speedup 1.0000x reference)
import jax
import jax.numpy as jnp
from jax.experimental import pallas as pl


def kernel(input):
    raise NotImplementedError("write your pallas kernel here")


# speedup vs baseline: 20.3866x; 20.3866x over previous
from math import ceil

import jax
import jax.numpy as jnp
from jax.experimental import pallas as pl

N_ROWS = 1048576
N_COLS = 16
K = ceil(0.1 * N_ROWS)
RS_ROWS = N_ROWS * N_COLS // 128
BLOCK_R = 2048
NB = RS_ROWS // BLOCK_R
NBUCKETS = 8
NLEVELS = 4


def _minmax_body(x_ref, o_ref):
    @pl.when(pl.program_id(0) == 0)
    def _():
        o_ref[0, :] = jnp.full((128,), jnp.inf, jnp.float32)
        o_ref[1, :] = jnp.full((128,), -jnp.inf, jnp.float32)

    v = x_ref[...]
    o_ref[0, :] = jnp.minimum(o_ref[0, :], jnp.min(v, axis=0))
    o_ref[1, :] = jnp.maximum(o_ref[1, :], jnp.max(v, axis=0))


def _hist_body(lo_ref, s_ref, x_ref, o_ref):
    @pl.when(pl.program_id(0) == 0)
    def _():
        o_ref[...] = jnp.zeros((NBUCKETS, 128), jnp.float32)

    v = x_ref[...]
    t = jnp.floor((v - lo_ref[0, :][None, :]) * s_ref[0, :][None, :])
    for b in range(NBUCKETS):
        m = (t == float(b)).astype(jnp.float32)
        o_ref[b, :] = o_ref[b, :] + jnp.sum(m, axis=0)


def _sum_body(t_ref, x_ref, o_ref):
    @pl.when(pl.program_id(0) == 0)
    def _():
        o_ref[...] = jnp.zeros((2, 128), jnp.float32)

    v = x_ref[...]
    m = v < t_ref[0, :][None, :]
    o_ref[0, :] = o_ref[0, :] + jnp.sum(jnp.where(m, v, 0.0), axis=0)
    o_ref[1, :] = o_ref[1, :] + jnp.sum(m.astype(jnp.float32), axis=0)


def _lanes_to_cols(a):
    return a.reshape(a.shape[0], 8, 16).sum(axis=1)


def kernel(input):
    x = input.reshape(RS_ROWS, 128)
    grid = (NB,)
    xspec = pl.BlockSpec((BLOCK_R, 128), lambda i: (i, 0))
    small = lambda r: pl.BlockSpec((r, 128), lambda i: (0, 0))

    mm = pl.pallas_call(
        _minmax_body,
        grid=grid,
        in_specs=[xspec],
        out_specs=small(2),
        out_shape=jax.ShapeDtypeStruct((2, 128), jnp.float32),
    )(x)
    mm_c = mm.reshape(2, 8, 16)
    lo = mm_c[0].min(axis=0)
    hi = mm_c[1].max(axis=0)
    width = jnp.maximum(hi - lo, 1e-30) / NBUCKETS
    below = jnp.zeros((16,), jnp.float32)

    hist_call = pl.pallas_call(
        _hist_body,
        grid=grid,
        in_specs=[small(1), small(1), xspec],
        out_specs=small(NBUCKETS),
        out_shape=jax.ShapeDtypeStruct((NBUCKETS, 128), jnp.float32),
    )

    for _ in range(NLEVELS):
        lo128 = jnp.tile(lo, 8)[None, :]
        s128 = jnp.tile(1.0 / width, 8)[None, :]
        h = hist_call(lo128, s128, x)
        cnt = _lanes_to_cols(h)
        cum = jnp.cumsum(cnt, axis=0)
        need = K - below
        j = jnp.minimum(
            jnp.sum((cum < need[None, :]).astype(jnp.int32), axis=0), NBUCKETS - 1
        ).astype(jnp.float32)
        prev = jnp.concatenate([jnp.zeros((1, 16), jnp.float32), cum[:-1]], axis=0)
        below = below + jnp.take_along_axis(
            prev, j.astype(jnp.int32)[None, :], axis=0
        )[0]
        lo = lo + j * width
        width = width / NBUCKETS

    t_star = lo
    sc = pl.pallas_call(
        _sum_body,
        grid=grid,
        in_specs=[small(1), xspec],
        out_specs=small(2),
        out_shape=jax.ShapeDtypeStruct((2, 128), jnp.float32),
    )(jnp.tile(t_star, 8)[None, :], x)
    sc_c = sc.reshape(2, 8, 16).sum(axis=1)
    sum_below, cnt_below = sc_c[0], sc_c[1]
    r = K - cnt_below
    approx = t_star + (width * NBUCKETS) * 0.5
    return -(sum_below + r * approx) / K


# speedup vs baseline: 20.7503x; 1.0178x over previous
from math import ceil

import jax
import jax.numpy as jnp
from jax.experimental import pallas as pl
from jax.experimental.pallas import tpu as pltpu

N_ROWS = 1048576
N_COLS = 16
K = ceil(0.1 * N_ROWS)
RS_ROWS = N_ROWS * N_COLS // 128
BLOCK_R = 2048
NB = RS_ROWS // BLOCK_R
NBUCK = 8
NLEVELS = 4
NPASS = NLEVELS + 2


def _fold(t):
    def red(t, op):
        for sh in (16, 32, 64):
            t = op(t, pltpu.roll(t, sh, 1))
        return t

    return red


def _body(x_ref, o_ref, mn, mx, lo, w, below, hist, sacc):
    p = pl.program_id(0)
    i = pl.program_id(1)
    v = x_ref[...]

    @pl.when(p == 0)
    def _():
        @pl.when(i == 0)
        def _():
            mn[...] = jnp.full((1, 128), jnp.inf, jnp.float32)
            mx[...] = jnp.full((1, 128), -jnp.inf, jnp.float32)

        mn[...] = jnp.minimum(mn[...], jnp.min(v, axis=0, keepdims=True))
        mx[...] = jnp.maximum(mx[...], jnp.max(v, axis=0, keepdims=True))

        @pl.when(i == NB - 1)
        def _():
            m = mn[...]
            M = mx[...]
            for sh in (16, 32, 64):
                m = jnp.minimum(m, pltpu.roll(m, sh, 1))
                M = jnp.maximum(M, pltpu.roll(M, sh, 1))
            lo[...] = m
            w[...] = jnp.maximum(M - m, 1e-30) / NBUCK
            below[...] = jnp.zeros((1, 128), jnp.float32)

    @pl.when((p >= 1) & (p <= NLEVELS))
    def _():
        @pl.when(i == 0)
        def _():
            hist[...] = jnp.zeros((NBUCK, 128), jnp.float32)

        t = jnp.floor((v - lo[...]) * (1.0 / w[...]))
        for b in range(NBUCK):
            m = jnp.where(t == float(b), 1.0, 0.0)
            hist[b, :] = hist[b, :] + jnp.sum(m, axis=0)

        @pl.when(i == NB - 1)
        def _():
            need = float(K) - below[...]
            cum = jnp.zeros((1, 128), jnp.float32)
            j = jnp.zeros((1, 128), jnp.float32)
            add = jnp.zeros((1, 128), jnp.float32)
            for b in range(NBUCK - 1):
                cnt_b = hist[b, :][None, :]
                for sh in (16, 32, 64):
                    cnt_b = cnt_b + pltpu.roll(cnt_b, sh, 1)
                cum = cum + cnt_b
                sel = jnp.where(cum < need, 1.0, 0.0)
                j = j + sel
                add = add + sel * cnt_b
            below[...] = below[...] + add
            lo[...] = lo[...] + j * w[...]
            w[...] = w[...] / NBUCK

    @pl.when(p == NPASS - 1)
    def _():
        @pl.when(i == 0)
        def _():
            sacc[...] = jnp.zeros((2, 128), jnp.float32)

        msk = v < lo[...]
        sacc[0, :] = sacc[0, :] + jnp.sum(jnp.where(msk, v, 0.0), axis=0)
        sacc[1, :] = sacc[1, :] + jnp.sum(jnp.where(msk, 1.0, 0.0), axis=0)

        @pl.when(i == NB - 1)
        def _():
            s = sacc[0, :][None, :]
            c = sacc[1, :][None, :]
            for sh in (16, 32, 64):
                s = s + pltpu.roll(s, sh, 1)
                c = c + pltpu.roll(c, sh, 1)
            r = float(K) - c
            approx = lo[...] + w[...] * (NBUCK * 0.5)
            o_ref[...] = -(s + r * approx) * (1.0 / K)


def kernel(input):
    x = input.reshape(RS_ROWS, 128)
    out = pl.pallas_call(
        _body,
        grid=(NPASS, NB),
        in_specs=[pl.BlockSpec((BLOCK_R, 128), lambda p, i: (i, 0))],
        out_specs=pl.BlockSpec((1, 128), lambda p, i: (0, 0)),
        out_shape=jax.ShapeDtypeStruct((1, 128), jnp.float32),
        scratch_shapes=[
            pltpu.VMEM((1, 128), jnp.float32),
            pltpu.VMEM((1, 128), jnp.float32),
            pltpu.VMEM((1, 128), jnp.float32),
            pltpu.VMEM((1, 128), jnp.float32),
            pltpu.VMEM((1, 128), jnp.float32),
            pltpu.VMEM((NBUCK, 128), jnp.float32),
            pltpu.VMEM((2, 128), jnp.float32),
        ],
    )(x)
    return out[0, :N_COLS]
